# baseline (speedup 1.0000x reference)
import functools

import jax
import jax.numpy as jnp
from jax import lax
from jax.experimental import pallas as pl
from jax.experimental.pallas import tpu as pltpu
from jax.experimental.pallas import tpu_sc as plsc

D = 1024
H = 1024
O = 1024
N = 2048
MG = 2
MAC = 3
MIC = 4
G = MG * MAC
ALPHA = 0.01

B = 256
KMAX = N // B
NBMAX = N // B + G
NPAD = N + G * B

_NC, _NS = 2, 16
_NW = _NC * _NS
_SLOTS_W = NPAD // _NW
_TOKS_W = N // _NW


def _router_body(x_ref, w_ref, b_ref, ids_ref, aux_ref):
    x = x_ref[...]
    logits = jnp.dot(x, w_ref[...], preferred_element_type=jnp.float32)
    logits = logits + b_ref[...]
    nf = jnp.float32(N)

    a0 = logits[:, 0:1]
    a1 = logits[:, 1:2]
    mx = jnp.maximum(a0, a1)
    e0 = jnp.exp(a0 - mx)
    e1 = jnp.exp(a1 - mx)
    s = e0 + e1
    f0 = jnp.sum(e0 / s) / nf
    f1 = jnp.sum(e1 / s) / nf
    aux = ALPHA * 2.0 * (f0 * f0 + f1 * f1)

    topi = (a1 > a0).astype(jnp.int32)
    msel = [None, None]
    for g in range(MG):
        base = MG + MAC * g
        c0 = logits[:, base:base + 1]
        c1 = logits[:, base + 1:base + 2]
        c2 = logits[:, base + 2:base + 3]
        m = jnp.maximum(jnp.maximum(c0, c1), c2)
        x0 = jnp.exp(c0 - m)
        x1 = jnp.exp(c1 - m)
        x2 = jnp.exp(c2 - m)
        ssum = x0 + x1 + x2
        maskg = (topi == g).astype(jnp.float32)
        cnt = jnp.sum(maskg)
        denom = jnp.maximum(cnt, 1.0)
        fj0 = jnp.sum(x0 / ssum * maskg) / denom
        fj1 = jnp.sum(x1 / ssum * maskg) / denom
        fj2 = jnp.sum(x2 / ssum * maskg) / denom
        lb = ALPHA * 3.0 * (fj0 * fj0 + fj1 * fj1 + fj2 * fj2)
        aux = aux + jnp.where(cnt > 0.0, lb, 0.0)
        msel[g] = jnp.where(c1 > c0,
                            jnp.where(c2 > c1, 2, 1),
                            jnp.where(c2 > c0, 2, 0)).astype(jnp.int32)

    ids = topi * MAC + jnp.where(topi == 1, msel[1], msel[0])
    ids_ref[...] = ids
    aux_ref[...] = jnp.full((8, 128), aux, jnp.float32)


def _router(x, wcat, bcat):
    return pl.pallas_call(
        _router_body,
        out_shape=[
            jax.ShapeDtypeStruct((N, 1), jnp.int32),
            jax.ShapeDtypeStruct((8, 128), jnp.float32),
        ],
    )(x, wcat, bcat)


@functools.lru_cache(maxsize=1)
def _sc_gathers():
    mesh = plsc.VectorSubcoreMesh(core_axis_name="c", subcore_axis_name="s",
                                  num_cores=_NC, num_subcores=_NS)

    @functools.partial(
        pl.kernel,
        out_type=jax.ShapeDtypeStruct((NPAD, D), jnp.float32),
        mesh=mesh,
        scratch_types=[
            pltpu.VMEM((_SLOTS_W,), jnp.int32),
            pltpu.VMEM((_SLOTS_W, D), jnp.float32),
            pltpu.SemaphoreType.DMA,
        ],
    )
    def gather_in(x_hbm, idx_hbm, out_hbm, idx_v, rows_v, sem):
        wid = lax.axis_index("s") * _NC + lax.axis_index("c")
        base = wid * _SLOTS_W
        pltpu.sync_copy(idx_hbm.at[pl.ds(base, _SLOTS_W)], idx_v)
        pltpu.async_copy(x_hbm.at[idx_v], rows_v, sem).wait()
        pltpu.sync_copy(rows_v, out_hbm.at[pl.ds(base, _SLOTS_W)])

    @functools.partial(
        pl.kernel,
        out_type=jax.ShapeDtypeStruct((N, O), jnp.float32),
        mesh=mesh,
        scratch_types=[
            pltpu.VMEM((_TOKS_W,), jnp.int32),
            pltpu.VMEM((_TOKS_W, O), jnp.float32),
            pltpu.SemaphoreType.DMA,
        ],
    )
    def gather_out(tab_hbm, idx_hbm, out_hbm, idx_v, rows_v, sem):
        wid = lax.axis_index("s") * _NC + lax.axis_index("c")
        base = wid * _TOKS_W
        pltpu.sync_copy(idx_hbm.at[pl.ds(base, _TOKS_W)], idx_v)
        pltpu.async_copy(tab_hbm.at[idx_v], rows_v, sem).wait()
        pltpu.sync_copy(rows_v, out_hbm.at[pl.ds(base, _TOKS_W)])

    return gather_in, gather_out


def _ffn_body(g_ref, r_ref, v_ref, xs_ref, w1_ref, b1_ref, w2_ref, b2_ref,
              w3_ref, b3_ref, out_ref):
    i = pl.program_id(0)
    e = pl.program_id(1)
    valid = v_ref[i] == 1

    @pl.when(valid)
    def _():
        xb = xs_ref[...]
        xb16 = xb.astype(jnp.bfloat16)
        h = jnp.dot(xb16, w1_ref[0, 0], preferred_element_type=jnp.float32)
        h = jnp.maximum(h + b1_ref[0, 0], 0.0)
        h2 = jnp.dot(h.astype(jnp.bfloat16), w2_ref[0, 0],
                     preferred_element_type=jnp.float32)
        h2 = jnp.maximum(h2 + b2_ref[0, 0] + xb, 0.0)
        oe = jnp.dot(h2.astype(jnp.bfloat16), w3_ref[0, 0],
                     preferred_element_type=jnp.float32)
        oe = (oe + b3_ref[0, 0]) * (1.0 / MIC)

        @pl.when(e == 0)
        def _():
            out_ref[...] = oe

        @pl.when(e > 0)
        def _():
            out_ref[...] += oe


def _ffn(blk_gid, blk_row, blk_val, xs_pad, fc1_W, b1r, fc2_W, b2r, fc3_W, b3r):
    def _e_eff(e, v, i):
        return jnp.where(v[i] == 1, e, MIC - 1)

    grid_spec = pltpu.PrefetchScalarGridSpec(
        num_scalar_prefetch=3,
        grid=(NBMAX, MIC),
        in_specs=[
            pl.BlockSpec((B, D), lambda i, e, g, r, v: (r[i], 0)),
            pl.BlockSpec((1, 1, D, H),
                         lambda i, e, g, r, v: (g[i], _e_eff(e, v, i), 0, 0)),
            pl.BlockSpec((1, 1, H),
                         lambda i, e, g, r, v: (g[i] * MIC + _e_eff(e, v, i), 0, 0)),
            pl.BlockSpec((1, 1, H, H),
                         lambda i, e, g, r, v: (g[i], _e_eff(e, v, i), 0, 0)),
            pl.BlockSpec((1, 1, H),
                         lambda i, e, g, r, v: (g[i] * MIC + _e_eff(e, v, i), 0, 0)),
            pl.BlockSpec((1, 1, H, O),
                         lambda i, e, g, r, v: (g[i], _e_eff(e, v, i), 0, 0)),
            pl.BlockSpec((1, 1, O),
                         lambda i, e, g, r, v: (g[i] * MIC + _e_eff(e, v, i), 0, 0)),
        ],
        out_specs=pl.BlockSpec((B, O), lambda i, e, g, r, v: (r[i], 0)),
    )
    return pl.pallas_call(
        _ffn_body,
        grid_spec=grid_spec,
        out_shape=jax.ShapeDtypeStruct((NPAD, O), jnp.float32),
        compiler_params=pltpu.CompilerParams(
            dimension_semantics=("arbitrary", "arbitrary")),
    )(blk_gid, blk_row, blk_val, xs_pad, fc1_W, b1r, fc2_W, b2r, fc3_W, b3r)


def kernel(x, meta_W, meta_b, macro_W, macro_b,
           fc1_W, fc1_b, fc2_W, fc2_b, fc3_W, fc3_b):
    wcat = jnp.concatenate(
        [meta_W, macro_W[0], macro_W[1],
         jnp.zeros((D, 128 - MG - MG * MAC), jnp.float32)], axis=1)
    bcat = jnp.concatenate(
        [meta_b, macro_b[0], macro_b[1],
         jnp.zeros((128 - MG - MG * MAC,), jnp.float32)])[None, :]

    ids2d, aux2d = _router(x, wcat, bcat)
    ids = ids2d[:, 0]
    aux = aux2d[0, 0]

    c6 = jnp.arange(G, dtype=jnp.int32)
    oh = (ids[:, None] == c6[None, :]).astype(jnp.int32)
    counts = jnp.sum(oh, axis=0)
    ranks = jnp.take_along_axis(jnp.cumsum(oh, axis=0) - 1,
                                ids[:, None], axis=1)[:, 0]
    nb = (counts + B - 1) // B
    astart = jnp.concatenate(
        [jnp.zeros((1,), jnp.int32), jnp.cumsum(nb * B)])[:G]
    p_tok = astart[ids] + ranks
    tok_for_slot = jnp.zeros((NPAD,), jnp.int32).at[p_tok].set(
        jnp.arange(N, dtype=jnp.int32))

    total_nb = jnp.sum(nb)
    cand_gid = jnp.repeat(c6, KMAX)
    cand_k = jnp.tile(jnp.arange(KMAX, dtype=jnp.int32), G)
    cand_valid = cand_k < nb[cand_gid]
    cand_row = astart[cand_gid] // B + cand_k
    order = jnp.argsort(jnp.logical_not(cand_valid), stable=True)
    g_s = cand_gid[order][:NBMAX]
    r_s = cand_row[order][:NBMAX]
    j = jnp.arange(NBMAX, dtype=jnp.int32)
    g_last = g_s[total_nb - 1]
    r_last = r_s[total_nb - 1]
    blk_gid = jnp.where(j < total_nb, g_s, g_last).astype(jnp.int32)
    blk_row = jnp.where(j < total_nb, r_s, r_last).astype(jnp.int32)
    blk_val = (j < total_nb).astype(jnp.int32)

    gather_in, gather_out = _sc_gathers()
    xs_pad = gather_in(x, tok_for_slot)

    b1r = fc1_b.reshape(G * MIC, 1, H)
    b2r = fc2_b.reshape(G * MIC, 1, H)
    b3r = fc3_b.reshape(G * MIC, 1, O)
    out_pad = _ffn(blk_gid, blk_row, blk_val, xs_pad,
                   fc1_W.astype(jnp.bfloat16), b1r,
                   fc2_W.astype(jnp.bfloat16), b2r,
                   fc3_W.astype(jnp.bfloat16), b3r)

    final = gather_out(out_pad, p_tok)
    return final, aux


# speedup vs baseline: 1.5177x; 1.5177x over previous
import functools

import jax
import jax.numpy as jnp
from jax import lax
from jax.experimental import pallas as pl
from jax.experimental.pallas import tpu as pltpu
from jax.experimental.pallas import tpu_sc as plsc

D = 1024
H = 1024
O = 1024
N = 2048
MG = 2
MAC = 3
MIC = 4
G = MG * MAC
ALPHA = 0.01

B = 256
KMAX = N // B
NBMAX = N // B + G
NPAD = N + G * B

_NC, _NS = 2, 16
_NW = _NC * _NS
_SLOTS_W = NPAD // _NW
_TOKS_W = N // _NW


def _router_body(x_ref, w_ref, b_ref, ids_ref, aux_ref):
    x = x_ref[...]
    logits = jnp.dot(x, w_ref[...], preferred_element_type=jnp.float32)
    logits = logits + b_ref[...]
    nf = jnp.float32(N)

    a0 = logits[:, 0:1]
    a1 = logits[:, 1:2]
    mx = jnp.maximum(a0, a1)
    e0 = jnp.exp(a0 - mx)
    e1 = jnp.exp(a1 - mx)
    s = e0 + e1
    f0 = jnp.sum(e0 / s) / nf
    f1 = jnp.sum(e1 / s) / nf
    aux = ALPHA * 2.0 * (f0 * f0 + f1 * f1)

    topi = (a1 > a0).astype(jnp.int32)
    msel = [None, None]
    for g in range(MG):
        base = MG + MAC * g
        c0 = logits[:, base:base + 1]
        c1 = logits[:, base + 1:base + 2]
        c2 = logits[:, base + 2:base + 3]
        m = jnp.maximum(jnp.maximum(c0, c1), c2)
        x0 = jnp.exp(c0 - m)
        x1 = jnp.exp(c1 - m)
        x2 = jnp.exp(c2 - m)
        ssum = x0 + x1 + x2
        maskg = (topi == g).astype(jnp.float32)
        cnt = jnp.sum(maskg)
        denom = jnp.maximum(cnt, 1.0)
        fj0 = jnp.sum(x0 / ssum * maskg) / denom
        fj1 = jnp.sum(x1 / ssum * maskg) / denom
        fj2 = jnp.sum(x2 / ssum * maskg) / denom
        lb = ALPHA * 3.0 * (fj0 * fj0 + fj1 * fj1 + fj2 * fj2)
        aux = aux + jnp.where(cnt > 0.0, lb, 0.0)
        msel[g] = jnp.where(c1 > c0,
                            jnp.where(c2 > c1, 2, 1),
                            jnp.where(c2 > c0, 2, 0)).astype(jnp.int32)

    ids = topi * MAC + jnp.where(topi == 1, msel[1], msel[0])
    ids_ref[...] = ids
    aux_ref[...] = jnp.full((8, 128), aux, jnp.float32)


def _router(x, wcat, bcat):
    return pl.pallas_call(
        _router_body,
        out_shape=[
            jax.ShapeDtypeStruct((N, 1), jnp.int32),
            jax.ShapeDtypeStruct((8, 128), jnp.float32),
        ],
    )(x, wcat, bcat)


@functools.lru_cache(maxsize=1)
def _sc_gathers():
    mesh = plsc.VectorSubcoreMesh(core_axis_name="c", subcore_axis_name="s",
                                  num_cores=_NC, num_subcores=_NS)

    @functools.partial(
        pl.kernel,
        out_type=jax.ShapeDtypeStruct((NPAD, D), jnp.float32),
        mesh=mesh,
        scratch_types=[
            pltpu.VMEM((_SLOTS_W,), jnp.int32),
            pltpu.VMEM((_SLOTS_W, D), jnp.float32),
            pltpu.SemaphoreType.DMA,
        ],
    )
    def gather_in(x_hbm, idx_hbm, out_hbm, idx_v, rows_v, sem):
        wid = lax.axis_index("s") * _NC + lax.axis_index("c")
        base = wid * _SLOTS_W
        pltpu.sync_copy(idx_hbm.at[pl.ds(base, _SLOTS_W)], idx_v)
        pltpu.async_copy(x_hbm.at[idx_v], rows_v, sem).wait()
        pltpu.sync_copy(rows_v, out_hbm.at[pl.ds(base, _SLOTS_W)])

    @functools.partial(
        pl.kernel,
        out_type=jax.ShapeDtypeStruct((N, O), jnp.float32),
        mesh=mesh,
        scratch_types=[
            pltpu.VMEM((_TOKS_W,), jnp.int32),
            pltpu.VMEM((_TOKS_W, O), jnp.float32),
            pltpu.SemaphoreType.DMA,
        ],
    )
    def gather_out(tab_hbm, idx_hbm, out_hbm, idx_v, rows_v, sem):
        wid = lax.axis_index("s") * _NC + lax.axis_index("c")
        base = wid * _TOKS_W
        pltpu.sync_copy(idx_hbm.at[pl.ds(base, _TOKS_W)], idx_v)
        pltpu.async_copy(tab_hbm.at[idx_v], rows_v, sem).wait()
        pltpu.sync_copy(rows_v, out_hbm.at[pl.ds(base, _TOKS_W)])

    return gather_in, gather_out


def _ffn_body(g_ref, r_ref, v_ref, xs_ref, w1_ref, b1_ref, w2_ref, b2_ref,
              w3_ref, b3_ref, out_ref):
    i = pl.program_id(0)
    e = pl.program_id(1)
    valid = v_ref[i] == 1

    @pl.when(valid)
    def _():
        xb = xs_ref[...]
        h = jnp.dot(xb, w1_ref[0, 0], preferred_element_type=jnp.float32)
        h = jnp.maximum(h + b1_ref[0, 0], 0.0)
        h2 = jnp.dot(h, w2_ref[0, 0], preferred_element_type=jnp.float32)
        h2 = jnp.maximum(h2 + b2_ref[0, 0] + xb, 0.0)
        oe = jnp.dot(h2, w3_ref[0, 0], preferred_element_type=jnp.float32)
        oe = (oe + b3_ref[0, 0]) * (1.0 / MIC)

        @pl.when(e == 0)
        def _():
            out_ref[...] = oe

        @pl.when(e > 0)
        def _():
            out_ref[...] += oe


def _ffn(blk_gid, blk_row, blk_val, xs_pad, fc1_W, b1r, fc2_W, b2r, fc3_W, b3r):
    def _e_eff(e, v, i):
        return jnp.where(v[i] == 1, e, MIC - 1)

    grid_spec = pltpu.PrefetchScalarGridSpec(
        num_scalar_prefetch=3,
        grid=(NBMAX, MIC),
        in_specs=[
            pl.BlockSpec((B, D), lambda i, e, g, r, v: (r[i], 0)),
            pl.BlockSpec((1, 1, D, H),
                         lambda i, e, g, r, v: (g[i], _e_eff(e, v, i), 0, 0)),
            pl.BlockSpec((1, 1, H),
                         lambda i, e, g, r, v: (g[i] * MIC + _e_eff(e, v, i), 0, 0)),
            pl.BlockSpec((1, 1, H, H),
                         lambda i, e, g, r, v: (g[i], _e_eff(e, v, i), 0, 0)),
            pl.BlockSpec((1, 1, H),
                         lambda i, e, g, r, v: (g[i] * MIC + _e_eff(e, v, i), 0, 0)),
            pl.BlockSpec((1, 1, H, O),
                         lambda i, e, g, r, v: (g[i], _e_eff(e, v, i), 0, 0)),
            pl.BlockSpec((1, 1, O),
                         lambda i, e, g, r, v: (g[i] * MIC + _e_eff(e, v, i), 0, 0)),
        ],
        out_specs=pl.BlockSpec((B, O), lambda i, e, g, r, v: (r[i], 0)),
    )
    return pl.pallas_call(
        _ffn_body,
        grid_spec=grid_spec,
        out_shape=jax.ShapeDtypeStruct((NPAD, O), jnp.float32),
        compiler_params=pltpu.CompilerParams(
            dimension_semantics=("arbitrary", "arbitrary")),
    )(blk_gid, blk_row, blk_val, xs_pad, fc1_W, b1r, fc2_W, b2r, fc3_W, b3r)


def kernel(x, meta_W, meta_b, macro_W, macro_b,
           fc1_W, fc1_b, fc2_W, fc2_b, fc3_W, fc3_b):
    wcat = jnp.concatenate(
        [meta_W, macro_W[0], macro_W[1],
         jnp.zeros((D, 128 - MG - MG * MAC), jnp.float32)], axis=1)
    bcat = jnp.concatenate(
        [meta_b, macro_b[0], macro_b[1],
         jnp.zeros((128 - MG - MG * MAC,), jnp.float32)])[None, :]

    ids2d, aux2d = _router(x, wcat, bcat)
    ids = ids2d[:, 0]
    aux = aux2d[0, 0]

    c6 = jnp.arange(G, dtype=jnp.int32)
    oh = (ids[:, None] == c6[None, :]).astype(jnp.int32)
    counts = jnp.sum(oh, axis=0)
    ranks = jnp.take_along_axis(jnp.cumsum(oh, axis=0) - 1,
                                ids[:, None], axis=1)[:, 0]
    nb = (counts + B - 1) // B
    astart = jnp.concatenate(
        [jnp.zeros((1,), jnp.int32), jnp.cumsum(nb * B)])[:G]
    p_tok = astart[ids] + ranks
    tok_for_slot = (jnp.arange(NPAD, dtype=jnp.int32) % N).at[p_tok].set(
        jnp.arange(N, dtype=jnp.int32))

    total_nb = jnp.sum(nb)
    cand_gid = jnp.repeat(c6, KMAX)
    cand_k = jnp.tile(jnp.arange(KMAX, dtype=jnp.int32), G)
    cand_valid = cand_k < nb[cand_gid]
    cand_row = astart[cand_gid] // B + cand_k
    order = jnp.argsort(jnp.logical_not(cand_valid), stable=True)
    g_s = cand_gid[order][:NBMAX]
    r_s = cand_row[order][:NBMAX]
    j = jnp.arange(NBMAX, dtype=jnp.int32)
    g_last = g_s[total_nb - 1]
    r_last = r_s[total_nb - 1]
    blk_gid = jnp.where(j < total_nb, g_s, g_last).astype(jnp.int32)
    blk_row = jnp.where(j < total_nb, r_s, r_last).astype(jnp.int32)
    blk_val = (j < total_nb).astype(jnp.int32)

    gather_in, gather_out = _sc_gathers()
    xs_pad = gather_in(x, tok_for_slot)

    b1r = fc1_b.reshape(G * MIC, 1, H)
    b2r = fc2_b.reshape(G * MIC, 1, H)
    b3r = fc3_b.reshape(G * MIC, 1, O)
    out_pad = _ffn(blk_gid, blk_row, blk_val, xs_pad,
                   fc1_W, b1r, fc2_W, b2r, fc3_W, b3r)

    final = gather_out(out_pad, p_tok)
    return final, aux


# speedup vs baseline: 2.1200x; 1.3969x over previous
import functools

import jax
import jax.numpy as jnp
from jax import lax
from jax.experimental import pallas as pl
from jax.experimental.pallas import tpu as pltpu
from jax.experimental.pallas import tpu_sc as plsc

D = 1024
H = 1024
O = 1024
N = 2048
MG = 2
MAC = 3
MIC = 4
G = MG * MAC
ALPHA = 0.01

B = 512
KMAX = N // B
NBMAX = N // B + G
NPAD = N + G * B

_NC, _NS = 2, 16
_NW = _NC * _NS
_SLOTS_W = NPAD // _NW
_GCH = 2
_SLOTS_CH = _SLOTS_W // _GCH
_TOKS_W = N // _NW


def _router_body(x_ref, w_ref, b_ref, ids_ref, aux_ref):
    x = x_ref[...]
    logits = jnp.dot(x, w_ref[...], preferred_element_type=jnp.float32)
    logits = logits + b_ref[...]
    nf = jnp.float32(N)

    a0 = logits[:, 0:1]
    a1 = logits[:, 1:2]
    mx = jnp.maximum(a0, a1)
    e0 = jnp.exp(a0 - mx)
    e1 = jnp.exp(a1 - mx)
    s = e0 + e1
    f0 = jnp.sum(e0 / s) / nf
    f1 = jnp.sum(e1 / s) / nf
    aux = ALPHA * 2.0 * (f0 * f0 + f1 * f1)

    topi = (a1 > a0).astype(jnp.int32)
    msel = [None, None]
    for g in range(MG):
        base = MG + MAC * g
        c0 = logits[:, base:base + 1]
        c1 = logits[:, base + 1:base + 2]
        c2 = logits[:, base + 2:base + 3]
        m = jnp.maximum(jnp.maximum(c0, c1), c2)
        x0 = jnp.exp(c0 - m)
        x1 = jnp.exp(c1 - m)
        x2 = jnp.exp(c2 - m)
        ssum = x0 + x1 + x2
        maskg = (topi == g).astype(jnp.float32)
        cnt = jnp.sum(maskg)
        denom = jnp.maximum(cnt, 1.0)
        fj0 = jnp.sum(x0 / ssum * maskg) / denom
        fj1 = jnp.sum(x1 / ssum * maskg) / denom
        fj2 = jnp.sum(x2 / ssum * maskg) / denom
        lb = ALPHA * 3.0 * (fj0 * fj0 + fj1 * fj1 + fj2 * fj2)
        aux = aux + jnp.where(cnt > 0.0, lb, 0.0)
        msel[g] = jnp.where(c1 > c0,
                            jnp.where(c2 > c1, 2, 1),
                            jnp.where(c2 > c0, 2, 0)).astype(jnp.int32)

    ids = topi * MAC + jnp.where(topi == 1, msel[1], msel[0])
    ids_ref[...] = ids
    aux_ref[...] = jnp.full((8, 128), aux, jnp.float32)


def _router(x, wcat, bcat):
    return pl.pallas_call(
        _router_body,
        out_shape=[
            jax.ShapeDtypeStruct((N, 1), jnp.int32),
            jax.ShapeDtypeStruct((8, 128), jnp.float32),
        ],
    )(x, wcat, bcat)


@functools.lru_cache(maxsize=1)
def _sc_gathers():
    mesh = plsc.VectorSubcoreMesh(core_axis_name="c", subcore_axis_name="s",
                                  num_cores=_NC, num_subcores=_NS)

    @functools.partial(
        pl.kernel,
        out_type=jax.ShapeDtypeStruct((NPAD, D), jnp.float32),
        mesh=mesh,
        scratch_types=[
            pltpu.VMEM((_SLOTS_CH,), jnp.int32),
            pltpu.VMEM((_SLOTS_CH, D), jnp.float32),
            pltpu.SemaphoreType.DMA,
        ],
    )
    def gather_in(x_hbm, idx_hbm, out_hbm, idx_v, rows_v, sem):
        wid = lax.axis_index("s") * _NC + lax.axis_index("c")
        for ch in range(_GCH):
            base = wid * _SLOTS_W + ch * _SLOTS_CH
            pltpu.sync_copy(idx_hbm.at[pl.ds(base, _SLOTS_CH)], idx_v)
            pltpu.async_copy(x_hbm.at[idx_v], rows_v, sem).wait()
            pltpu.sync_copy(rows_v, out_hbm.at[pl.ds(base, _SLOTS_CH)])

    @functools.partial(
        pl.kernel,
        out_type=jax.ShapeDtypeStruct((N, O), jnp.float32),
        mesh=mesh,
        scratch_types=[
            pltpu.VMEM((_TOKS_W,), jnp.int32),
            pltpu.VMEM((_TOKS_W, O), jnp.float32),
            pltpu.SemaphoreType.DMA,
        ],
    )
    def gather_out(tab_hbm, idx_hbm, out_hbm, idx_v, rows_v, sem):
        wid = lax.axis_index("s") * _NC + lax.axis_index("c")
        base = wid * _TOKS_W
        pltpu.sync_copy(idx_hbm.at[pl.ds(base, _TOKS_W)], idx_v)
        pltpu.async_copy(tab_hbm.at[idx_v], rows_v, sem).wait()
        pltpu.sync_copy(rows_v, out_hbm.at[pl.ds(base, _TOKS_W)])

    return gather_in, gather_out


def _ffn_body(g_ref, r_ref, v_ref, xs_ref, w1_ref, b1_ref, w2_ref, b2_ref,
              w3_ref, b3_ref, out_ref):
    i = pl.program_id(0)
    e = pl.program_id(1)
    valid = v_ref[i] == 1

    @pl.when(valid)
    def _():
        xb = xs_ref[...]
        h = jnp.dot(xb, w1_ref[0, 0], preferred_element_type=jnp.float32)
        h = jnp.maximum(h + b1_ref[0, 0], 0.0)
        h2 = jnp.dot(h, w2_ref[0, 0], preferred_element_type=jnp.float32)
        h2 = jnp.maximum(h2 + b2_ref[0, 0] + xb, 0.0)
        oe = jnp.dot(h2, w3_ref[0, 0], preferred_element_type=jnp.float32)
        oe = (oe + b3_ref[0, 0]) * (1.0 / MIC)

        @pl.when(e == 0)
        def _():
            out_ref[...] = oe

        @pl.when(e > 0)
        def _():
            out_ref[...] += oe


def _ffn(blk_gid, blk_row, blk_val, xs_pad, fc1_W, b1r, fc2_W, b2r, fc3_W, b3r):
    def _e_eff(e, v, i):
        return jnp.where(v[i] == 1, e, MIC - 1)

    grid_spec = pltpu.PrefetchScalarGridSpec(
        num_scalar_prefetch=3,
        grid=(NBMAX, MIC),
        in_specs=[
            pl.BlockSpec((B, D), lambda i, e, g, r, v: (r[i], 0)),
            pl.BlockSpec((1, 1, D, H),
                         lambda i, e, g, r, v: (g[i], _e_eff(e, v, i), 0, 0)),
            pl.BlockSpec((1, 1, H),
                         lambda i, e, g, r, v: (g[i] * MIC + _e_eff(e, v, i), 0, 0)),
            pl.BlockSpec((1, 1, H, H),
                         lambda i, e, g, r, v: (g[i], _e_eff(e, v, i), 0, 0)),
            pl.BlockSpec((1, 1, H),
                         lambda i, e, g, r, v: (g[i] * MIC + _e_eff(e, v, i), 0, 0)),
            pl.BlockSpec((1, 1, H, O),
                         lambda i, e, g, r, v: (g[i], _e_eff(e, v, i), 0, 0)),
            pl.BlockSpec((1, 1, O),
                         lambda i, e, g, r, v: (g[i] * MIC + _e_eff(e, v, i), 0, 0)),
        ],
        out_specs=pl.BlockSpec((B, O), lambda i, e, g, r, v: (r[i], 0)),
    )
    return pl.pallas_call(
        _ffn_body,
        grid_spec=grid_spec,
        out_shape=jax.ShapeDtypeStruct((NPAD, O), jnp.float32),
        compiler_params=pltpu.CompilerParams(
            dimension_semantics=("arbitrary", "arbitrary")),
    )(blk_gid, blk_row, blk_val, xs_pad, fc1_W, b1r, fc2_W, b2r, fc3_W, b3r)


def kernel(x, meta_W, meta_b, macro_W, macro_b,
           fc1_W, fc1_b, fc2_W, fc2_b, fc3_W, fc3_b):
    wcat = jnp.concatenate(
        [meta_W, macro_W[0], macro_W[1],
         jnp.zeros((D, 128 - MG - MG * MAC), jnp.float32)], axis=1)
    bcat = jnp.concatenate(
        [meta_b, macro_b[0], macro_b[1],
         jnp.zeros((128 - MG - MG * MAC,), jnp.float32)])[None, :]

    ids2d, aux2d = _router(x, wcat, bcat)
    ids = ids2d[:, 0]
    aux = aux2d[0, 0]

    c6 = jnp.arange(G, dtype=jnp.int32)
    oh = (ids[:, None] == c6[None, :]).astype(jnp.int32)
    counts = jnp.sum(oh, axis=0)
    ranks = jnp.take_along_axis(jnp.cumsum(oh, axis=0) - 1,
                                ids[:, None], axis=1)[:, 0]
    nb = (counts + B - 1) // B
    astart = jnp.concatenate(
        [jnp.zeros((1,), jnp.int32), jnp.cumsum(nb * B)])[:G]
    p_tok = astart[ids] + ranks
    tok_for_slot = (jnp.arange(NPAD, dtype=jnp.int32) % N).at[p_tok].set(
        jnp.arange(N, dtype=jnp.int32))

    total_nb = jnp.sum(nb)
    cand_gid = jnp.repeat(c6, KMAX)
    cand_k = jnp.tile(jnp.arange(KMAX, dtype=jnp.int32), G)
    cand_valid = cand_k < nb[cand_gid]
    cand_row = astart[cand_gid] // B + cand_k
    order = jnp.argsort(jnp.logical_not(cand_valid), stable=True)
    g_s = cand_gid[order][:NBMAX]
    r_s = cand_row[order][:NBMAX]
    j = jnp.arange(NBMAX, dtype=jnp.int32)
    g_last = g_s[total_nb - 1]
    r_last = r_s[total_nb - 1]
    blk_gid = jnp.where(j < total_nb, g_s, g_last).astype(jnp.int32)
    blk_row = jnp.where(j < total_nb, r_s, r_last).astype(jnp.int32)
    blk_val = (j < total_nb).astype(jnp.int32)

    gather_in, gather_out = _sc_gathers()
    xs_pad = gather_in(x, tok_for_slot)

    b1r = fc1_b.reshape(G * MIC, 1, H)
    b2r = fc2_b.reshape(G * MIC, 1, H)
    b3r = fc3_b.reshape(G * MIC, 1, O)
    out_pad = _ffn(blk_gid, blk_row, blk_val, xs_pad,
                   fc1_W, b1r, fc2_W, b2r, fc3_W, b3r)

    final = gather_out(out_pad, p_tok)
    return final, aux


# speedup vs baseline: 2.4201x; 1.1416x over previous
import functools

import jax
import jax.numpy as jnp
from jax import lax
from jax.experimental import pallas as pl
from jax.experimental.pallas import tpu as pltpu
from jax.experimental.pallas import tpu_sc as plsc

D = 1024
H = 1024
O = 1024
N = 2048
MG = 2
MAC = 3
MIC = 4
G = MG * MAC
ALPHA = 0.01

B = 512
KMAX = N // B
NBMAX = N // B + G
NPAD = N + G * B

_NC, _NS = 2, 16
_NW = _NC * _NS
_SLOTS_W = NPAD // _NW
_GCH = 2
_SLOTS_CH = _SLOTS_W // _GCH
_TOKS_W = N // _NW


def _router_body(x_ref, w_ref, b_ref, ptok_ref, cnt_ref, aux_ref):
    x = x_ref[...]
    logits = jnp.dot(x, w_ref[...], preferred_element_type=jnp.float32)
    logits = logits + b_ref[...]
    nf = jnp.float32(N)

    a0 = logits[:, 0:1]
    a1 = logits[:, 1:2]
    mx = jnp.maximum(a0, a1)
    e0 = jnp.exp(a0 - mx)
    e1 = jnp.exp(a1 - mx)
    s = e0 + e1
    f0 = jnp.sum(e0 / s) / nf
    f1 = jnp.sum(e1 / s) / nf
    aux = ALPHA * 2.0 * (f0 * f0 + f1 * f1)

    topi = (a1 > a0).astype(jnp.int32)
    msel = [None, None]
    for g in range(MG):
        base = MG + MAC * g
        c0 = logits[:, base:base + 1]
        c1 = logits[:, base + 1:base + 2]
        c2 = logits[:, base + 2:base + 3]
        m = jnp.maximum(jnp.maximum(c0, c1), c2)
        x0 = jnp.exp(c0 - m)
        x1 = jnp.exp(c1 - m)
        x2 = jnp.exp(c2 - m)
        ssum = x0 + x1 + x2
        maskg = (topi == g).astype(jnp.float32)
        cnt = jnp.sum(maskg)
        denom = jnp.maximum(cnt, 1.0)
        fj0 = jnp.sum(x0 / ssum * maskg) / denom
        fj1 = jnp.sum(x1 / ssum * maskg) / denom
        fj2 = jnp.sum(x2 / ssum * maskg) / denom
        lb = ALPHA * 3.0 * (fj0 * fj0 + fj1 * fj1 + fj2 * fj2)
        aux = aux + jnp.where(cnt > 0.0, lb, 0.0)
        msel[g] = jnp.where(c1 > c0,
                            jnp.where(c2 > c1, 2, 1),
                            jnp.where(c2 > c0, 2, 0)).astype(jnp.int32)

    ids = topi * MAC + jnp.where(topi == 1, msel[1], msel[0])

    lane = lax.broadcasted_iota(jnp.int32, (N, 8), 1)
    oh = (ids == lane).astype(jnp.float32)
    s = oh
    k = 1
    while k < N:
        s = s + jnp.concatenate([jnp.zeros((k, 8), jnp.float32), s[:N - k]],
                                axis=0)
        k *= 2
    counts = s[N - 1:N, :]
    ranks = jnp.sum(oh * (s - 1.0), axis=1, keepdims=True)
    acap = jnp.ceil(counts * (1.0 / B)) * B
    ac = acap
    for kk in (1, 2, 4):
        ac = ac + jnp.concatenate(
            [jnp.zeros((1, kk), jnp.float32), ac[:, :8 - kk]], axis=1)
    astart = ac - acap
    base = jnp.sum(oh * astart, axis=1, keepdims=True)
    ptok_ref[...] = (base + ranks).astype(jnp.int32)
    cnt_ref[...] = counts.astype(jnp.int32)
    aux_ref[...] = jnp.full((8, 128), aux, jnp.float32)


def _router(x, wcat, bcat):
    return pl.pallas_call(
        _router_body,
        out_shape=[
            jax.ShapeDtypeStruct((N, 1), jnp.int32),
            jax.ShapeDtypeStruct((1, 8), jnp.int32),
            jax.ShapeDtypeStruct((8, 128), jnp.float32),
        ],
    )(x, wcat, bcat)


@functools.lru_cache(maxsize=1)
def _sc_gathers():
    mesh = plsc.VectorSubcoreMesh(core_axis_name="c", subcore_axis_name="s",
                                  num_cores=_NC, num_subcores=_NS)

    @functools.partial(
        pl.kernel,
        out_type=jax.ShapeDtypeStruct((NPAD, D), jnp.float32),
        mesh=mesh,
        scratch_types=[
            pltpu.VMEM((_TOKS_W,), jnp.int32),
            pltpu.VMEM((_TOKS_W, D), jnp.float32),
            pltpu.SemaphoreType.DMA,
        ],
    )
    def scatter_in(x_hbm, idx_hbm, out_hbm, idx_v, rows_v, sem):
        wid = lax.axis_index("s") * _NC + lax.axis_index("c")
        base = wid * _TOKS_W
        pltpu.sync_copy(idx_hbm.at[pl.ds(base, _TOKS_W)], idx_v)
        pltpu.sync_copy(x_hbm.at[pl.ds(base, _TOKS_W)], rows_v)
        pltpu.async_copy(rows_v, out_hbm.at[idx_v], sem).wait()

    @functools.partial(
        pl.kernel,
        out_type=jax.ShapeDtypeStruct((N, O), jnp.float32),
        mesh=mesh,
        scratch_types=[
            pltpu.VMEM((_TOKS_W,), jnp.int32),
            pltpu.VMEM((_TOKS_W, O), jnp.float32),
            pltpu.SemaphoreType.DMA,
        ],
    )
    def gather_out(tab_hbm, idx_hbm, out_hbm, idx_v, rows_v, sem):
        wid = lax.axis_index("s") * _NC + lax.axis_index("c")
        base = wid * _TOKS_W
        pltpu.sync_copy(idx_hbm.at[pl.ds(base, _TOKS_W)], idx_v)
        pltpu.async_copy(tab_hbm.at[idx_v], rows_v, sem).wait()
        pltpu.sync_copy(rows_v, out_hbm.at[pl.ds(base, _TOKS_W)])

    return scatter_in, gather_out


def _ffn_body(g_ref, r_ref, v_ref, xs_ref, w1_ref, b1_ref, w2_ref, b2_ref,
              w3_ref, b3_ref, out_ref):
    i = pl.program_id(0)
    e = pl.program_id(1)
    valid = v_ref[i] == 1

    @pl.when(valid)
    def _():
        xb = xs_ref[...]
        h = jnp.dot(xb, w1_ref[0, 0], preferred_element_type=jnp.float32)
        h = jnp.maximum(h + b1_ref[0, 0], 0.0)
        h2 = jnp.dot(h, w2_ref[0, 0], preferred_element_type=jnp.float32)
        h2 = jnp.maximum(h2 + b2_ref[0, 0] + xb, 0.0)
        oe = jnp.dot(h2, w3_ref[0, 0], preferred_element_type=jnp.float32)
        oe = (oe + b3_ref[0, 0]) * (1.0 / MIC)

        @pl.when(e == 0)
        def _():
            out_ref[...] = oe

        @pl.when(e > 0)
        def _():
            out_ref[...] += oe


def _ffn(blk_gid, blk_row, blk_val, xs_pad, fc1_W, b1r, fc2_W, b2r, fc3_W, b3r):
    def _e_eff(e, v, i):
        return jnp.where(v[i] == 1, e, MIC - 1)

    grid_spec = pltpu.PrefetchScalarGridSpec(
        num_scalar_prefetch=3,
        grid=(NBMAX, MIC),
        in_specs=[
            pl.BlockSpec((B, D), lambda i, e, g, r, v: (r[i], 0)),
            pl.BlockSpec((1, 1, D, H),
                         lambda i, e, g, r, v: (g[i], _e_eff(e, v, i), 0, 0)),
            pl.BlockSpec((1, 1, H),
                         lambda i, e, g, r, v: (g[i] * MIC + _e_eff(e, v, i), 0, 0)),
            pl.BlockSpec((1, 1, H, H),
                         lambda i, e, g, r, v: (g[i], _e_eff(e, v, i), 0, 0)),
            pl.BlockSpec((1, 1, H),
                         lambda i, e, g, r, v: (g[i] * MIC + _e_eff(e, v, i), 0, 0)),
            pl.BlockSpec((1, 1, H, O),
                         lambda i, e, g, r, v: (g[i], _e_eff(e, v, i), 0, 0)),
            pl.BlockSpec((1, 1, O),
                         lambda i, e, g, r, v: (g[i] * MIC + _e_eff(e, v, i), 0, 0)),
        ],
        out_specs=pl.BlockSpec((B, O), lambda i, e, g, r, v: (r[i], 0)),
    )
    return pl.pallas_call(
        _ffn_body,
        grid_spec=grid_spec,
        out_shape=jax.ShapeDtypeStruct((NPAD, O), jnp.float32),
        compiler_params=pltpu.CompilerParams(
            dimension_semantics=("arbitrary", "arbitrary")),
    )(blk_gid, blk_row, blk_val, xs_pad, fc1_W, b1r, fc2_W, b2r, fc3_W, b3r)


def kernel(x, meta_W, meta_b, macro_W, macro_b,
           fc1_W, fc1_b, fc2_W, fc2_b, fc3_W, fc3_b):
    wcat = jnp.concatenate(
        [meta_W, macro_W[0], macro_W[1],
         jnp.zeros((D, 128 - MG - MG * MAC), jnp.float32)], axis=1)
    bcat = jnp.concatenate(
        [meta_b, macro_b[0], macro_b[1],
         jnp.zeros((128 - MG - MG * MAC,), jnp.float32)])[None, :]

    ptok2d, cnt2d, aux2d = _router(x, wcat, bcat)
    p_tok = ptok2d[:, 0]
    aux = aux2d[0, 0]

    c6 = jnp.arange(G, dtype=jnp.int32)
    counts = cnt2d[0, :G]
    nb = (counts + B - 1) // B
    astart = jnp.concatenate(
        [jnp.zeros((1,), jnp.int32), jnp.cumsum(nb * B)])[:G]

    total_nb = jnp.sum(nb)
    cand_gid = jnp.repeat(c6, KMAX)
    cand_k = jnp.tile(jnp.arange(KMAX, dtype=jnp.int32), G)
    cand_valid = cand_k < nb[cand_gid]
    cand_row = astart[cand_gid] // B + cand_k
    order = jnp.argsort(jnp.logical_not(cand_valid), stable=True)
    g_s = cand_gid[order][:NBMAX]
    r_s = cand_row[order][:NBMAX]
    j = jnp.arange(NBMAX, dtype=jnp.int32)
    g_last = g_s[total_nb - 1]
    r_last = r_s[total_nb - 1]
    blk_gid = jnp.where(j < total_nb, g_s, g_last).astype(jnp.int32)
    blk_row = jnp.where(j < total_nb, r_s, r_last).astype(jnp.int32)
    blk_val = (j < total_nb).astype(jnp.int32)

    scatter_in, gather_out = _sc_gathers()
    xs_pad = scatter_in(x, p_tok)

    b1r = fc1_b.reshape(G * MIC, 1, H)
    b2r = fc2_b.reshape(G * MIC, 1, H)
    b3r = fc3_b.reshape(G * MIC, 1, O)
    out_pad = _ffn(blk_gid, blk_row, blk_val, xs_pad,
                   fc1_W, b1r, fc2_W, b2r, fc3_W, b3r)

    final = gather_out(out_pad, p_tok)
    return final, aux


# speedup vs baseline: 2.4891x; 1.0285x over previous
import functools

import jax
import jax.numpy as jnp
from jax import lax
from jax.experimental import pallas as pl
from jax.experimental.pallas import tpu as pltpu
from jax.experimental.pallas import tpu_sc as plsc

D = 1024
H = 1024
O = 1024
N = 2048
MG = 2
MAC = 3
MIC = 4
G = MG * MAC
ALPHA = 0.01

B = 512
KMAX = N // B
NBMAX = N // B + G - 1
NPAD = N + G * B

_NC, _NS = 2, 16
_NW = _NC * _NS
_SLOTS_W = NPAD // _NW
_GCH = 2
_SLOTS_CH = _SLOTS_W // _GCH
_TOKS_W = N // _NW


def _router_body(x_ref, w_ref, b_ref, ptok_ref, desc_ref, aux_ref):
    x = x_ref[...]
    logits = jnp.dot(x, w_ref[...], preferred_element_type=jnp.float32)
    logits = logits + b_ref[...]
    nf = jnp.float32(N)

    a0 = logits[:, 0:1]
    a1 = logits[:, 1:2]
    mx = jnp.maximum(a0, a1)
    e0 = jnp.exp(a0 - mx)
    e1 = jnp.exp(a1 - mx)
    s = e0 + e1
    f0 = jnp.sum(e0 / s) / nf
    f1 = jnp.sum(e1 / s) / nf
    aux = ALPHA * 2.0 * (f0 * f0 + f1 * f1)

    topi = (a1 > a0).astype(jnp.int32)
    msel = [None, None]
    for g in range(MG):
        base = MG + MAC * g
        c0 = logits[:, base:base + 1]
        c1 = logits[:, base + 1:base + 2]
        c2 = logits[:, base + 2:base + 3]
        m = jnp.maximum(jnp.maximum(c0, c1), c2)
        x0 = jnp.exp(c0 - m)
        x1 = jnp.exp(c1 - m)
        x2 = jnp.exp(c2 - m)
        ssum = x0 + x1 + x2
        maskg = (topi == g).astype(jnp.float32)
        cnt = jnp.sum(maskg)
        denom = jnp.maximum(cnt, 1.0)
        fj0 = jnp.sum(x0 / ssum * maskg) / denom
        fj1 = jnp.sum(x1 / ssum * maskg) / denom
        fj2 = jnp.sum(x2 / ssum * maskg) / denom
        lb = ALPHA * 3.0 * (fj0 * fj0 + fj1 * fj1 + fj2 * fj2)
        aux = aux + jnp.where(cnt > 0.0, lb, 0.0)
        msel[g] = jnp.where(c1 > c0,
                            jnp.where(c2 > c1, 2, 1),
                            jnp.where(c2 > c0, 2, 0)).astype(jnp.int32)

    ids = topi * MAC + jnp.where(topi == 1, msel[1], msel[0])

    lane = lax.broadcasted_iota(jnp.int32, (N, 8), 1)
    oh = (ids == lane).astype(jnp.float32)
    s = oh
    k = 1
    while k < N:
        s = s + jnp.concatenate([jnp.zeros((k, 8), jnp.float32), s[:N - k]],
                                axis=0)
        k *= 2
    counts = s[N - 1:N, :]
    ranks = jnp.sum(oh * (s - 1.0), axis=1, keepdims=True)
    nbf = jnp.ceil(counts * (1.0 / B))
    acap = nbf * B
    ac = acap
    cumnb = nbf
    for kk in (1, 2, 4):
        zz = jnp.zeros((1, kk), jnp.float32)
        ac = ac + jnp.concatenate([zz, ac[:, :8 - kk]], axis=1)
        cumnb = cumnb + jnp.concatenate([zz, cumnb[:, :8 - kk]], axis=1)
    astart = ac - acap
    base = jnp.sum(oh * astart, axis=1, keepdims=True)
    ptok_ref[...] = (base + ranks).astype(jnp.int32)

    lane8 = lax.broadcasted_iota(jnp.int32, (1, 8), 1)
    cum_c = [jnp.sum(jnp.where(lane8 == c, cumnb, 0.0)) for c in range(G)]
    exc_c = [jnp.sum(jnp.where(lane8 == c, cumnb - nbf, 0.0)) for c in range(G)]
    ast_c = [jnp.sum(jnp.where(lane8 == c, astart, 0.0)) * (1.0 / B)
             for c in range(G)]
    total_nb = jnp.sum(nbf)
    jlane = lax.broadcasted_iota(jnp.int32, (1, 16), 1).astype(jnp.float32)
    jc = jnp.minimum(jlane, total_nb - 1.0)
    gid = jnp.zeros((1, 16), jnp.float32)
    for c in range(G):
        gid = gid + jnp.where(jc >= cum_c[c], 1.0, 0.0)
    kj = jc
    rowj = jnp.zeros((1, 16), jnp.float32)
    for c in range(G):
        sel = (gid == c)
        kj = kj - jnp.where(sel, exc_c[c], 0.0)
        rowj = rowj + jnp.where(sel, ast_c[c], 0.0)
    desc_ref[...] = jnp.concatenate(
        [gid, rowj + kj, jnp.where(jlane < total_nb, 1.0, 0.0)],
        axis=0).astype(jnp.int32)
    aux_ref[...] = jnp.full((8, 128), aux, jnp.float32)


def _router(x, wcat, bcat):
    return pl.pallas_call(
        _router_body,
        out_shape=[
            jax.ShapeDtypeStruct((N, 1), jnp.int32),
            jax.ShapeDtypeStruct((3, 16), jnp.int32),
            jax.ShapeDtypeStruct((8, 128), jnp.float32),
        ],
    )(x, wcat, bcat)


@functools.lru_cache(maxsize=1)
def _sc_gathers():
    mesh = plsc.VectorSubcoreMesh(core_axis_name="c", subcore_axis_name="s",
                                  num_cores=_NC, num_subcores=_NS)

    @functools.partial(
        pl.kernel,
        out_type=jax.ShapeDtypeStruct((NPAD, D), jnp.float32),
        mesh=mesh,
        scratch_types=[
            pltpu.VMEM((_TOKS_W,), jnp.int32),
            pltpu.VMEM((_TOKS_W, D), jnp.float32),
            pltpu.SemaphoreType.DMA,
        ],
    )
    def scatter_in(x_hbm, idx_hbm, out_hbm, idx_v, rows_v, sem):
        wid = lax.axis_index("s") * _NC + lax.axis_index("c")
        base = wid * _TOKS_W
        pltpu.sync_copy(idx_hbm.at[pl.ds(base, _TOKS_W)], idx_v)
        pltpu.sync_copy(x_hbm.at[pl.ds(base, _TOKS_W)], rows_v)
        pltpu.async_copy(rows_v, out_hbm.at[idx_v], sem).wait()

    @functools.partial(
        pl.kernel,
        out_type=jax.ShapeDtypeStruct((N, O), jnp.float32),
        mesh=mesh,
        scratch_types=[
            pltpu.VMEM((_TOKS_W,), jnp.int32),
            pltpu.VMEM((_TOKS_W, O), jnp.float32),
            pltpu.SemaphoreType.DMA,
        ],
    )
    def gather_out(tab_hbm, idx_hbm, out_hbm, idx_v, rows_v, sem):
        wid = lax.axis_index("s") * _NC + lax.axis_index("c")
        base = wid * _TOKS_W
        pltpu.sync_copy(idx_hbm.at[pl.ds(base, _TOKS_W)], idx_v)
        pltpu.async_copy(tab_hbm.at[idx_v], rows_v, sem).wait()
        pltpu.sync_copy(rows_v, out_hbm.at[pl.ds(base, _TOKS_W)])

    return scatter_in, gather_out


def _ffn_body(g_ref, r_ref, v_ref, xs_ref, w1_ref, b1_ref, w2_ref, b2_ref,
              w3_ref, b3_ref, out_ref):
    i = pl.program_id(0)
    e = pl.program_id(1)
    valid = v_ref[i] == 1

    @pl.when(valid)
    def _():
        xb = xs_ref[...]
        h = jnp.dot(xb, w1_ref[0, 0], preferred_element_type=jnp.float32)
        h = jnp.maximum(h + b1_ref[0, 0], 0.0)
        h2 = jnp.dot(h, w2_ref[0, 0], preferred_element_type=jnp.float32)
        h2 = jnp.maximum(h2 + b2_ref[0, 0] + xb, 0.0)
        oe = jnp.dot(h2, w3_ref[0, 0], preferred_element_type=jnp.float32)
        oe = (oe + b3_ref[0, 0]) * (1.0 / MIC)

        @pl.when(e == 0)
        def _():
            out_ref[...] = oe

        @pl.when(e > 0)
        def _():
            out_ref[...] += oe


def _ffn(blk_gid, blk_row, blk_val, xs_pad, fc1_W, b1r, fc2_W, b2r, fc3_W, b3r):
    def _e_eff(e, v, i):
        return jnp.where(v[i] == 1, e, MIC - 1)

    grid_spec = pltpu.PrefetchScalarGridSpec(
        num_scalar_prefetch=3,
        grid=(NBMAX, MIC),
        in_specs=[
            pl.BlockSpec((B, D), lambda i, e, g, r, v: (r[i], 0)),
            pl.BlockSpec((1, 1, D, H),
                         lambda i, e, g, r, v: (g[i], _e_eff(e, v, i), 0, 0)),
            pl.BlockSpec((1, 1, H),
                         lambda i, e, g, r, v: (g[i] * MIC + _e_eff(e, v, i), 0, 0)),
            pl.BlockSpec((1, 1, H, H),
                         lambda i, e, g, r, v: (g[i], _e_eff(e, v, i), 0, 0)),
            pl.BlockSpec((1, 1, H),
                         lambda i, e, g, r, v: (g[i] * MIC + _e_eff(e, v, i), 0, 0)),
            pl.BlockSpec((1, 1, H, O),
                         lambda i, e, g, r, v: (g[i], _e_eff(e, v, i), 0, 0)),
            pl.BlockSpec((1, 1, O),
                         lambda i, e, g, r, v: (g[i] * MIC + _e_eff(e, v, i), 0, 0)),
        ],
        out_specs=pl.BlockSpec((B, O), lambda i, e, g, r, v: (r[i], 0)),
    )
    return pl.pallas_call(
        _ffn_body,
        grid_spec=grid_spec,
        out_shape=jax.ShapeDtypeStruct((NPAD, O), jnp.float32),
        compiler_params=pltpu.CompilerParams(
            dimension_semantics=("arbitrary", "arbitrary")),
    )(blk_gid, blk_row, blk_val, xs_pad, fc1_W, b1r, fc2_W, b2r, fc3_W, b3r)


def kernel(x, meta_W, meta_b, macro_W, macro_b,
           fc1_W, fc1_b, fc2_W, fc2_b, fc3_W, fc3_b):
    wcat = jnp.concatenate(
        [meta_W, macro_W[0], macro_W[1],
         jnp.zeros((D, 128 - MG - MG * MAC), jnp.float32)], axis=1)
    bcat = jnp.concatenate(
        [meta_b, macro_b[0], macro_b[1],
         jnp.zeros((128 - MG - MG * MAC,), jnp.float32)])[None, :]

    ptok2d, desc, aux2d = _router(x, wcat, bcat)
    p_tok = ptok2d[:, 0]
    aux = aux2d[0, 0]
    blk_gid = desc[0, :NBMAX]
    blk_row = desc[1, :NBMAX]
    blk_val = desc[2, :NBMAX]

    scatter_in, gather_out = _sc_gathers()
    xs_pad = scatter_in(x, p_tok)

    b1r = fc1_b.reshape(G * MIC, 1, H)
    b2r = fc2_b.reshape(G * MIC, 1, H)
    b3r = fc3_b.reshape(G * MIC, 1, O)
    out_pad = _ffn(blk_gid, blk_row, blk_val, xs_pad,
                   fc1_W, b1r, fc2_W, b2r, fc3_W, b3r)

    final = gather_out(out_pad, p_tok)
    return final, aux


# speedup vs baseline: 2.5768x; 1.0352x over previous
import functools

import jax
import jax.numpy as jnp
from jax import lax
from jax.experimental import pallas as pl
from jax.experimental.pallas import tpu as pltpu
from jax.experimental.pallas import tpu_sc as plsc

D = 1024
H = 1024
O = 1024
N = 2048
MG = 2
MAC = 3
MIC = 4
G = MG * MAC
ALPHA = 0.01

B = 512
KMAX = N // B
NBMAX = N // B + G - 1
NPAD = N + G * B

_NC, _NS = 2, 16
_NW = _NC * _NS
_SLOTS_W = NPAD // _NW
_GCH = 2
_SLOTS_CH = _SLOTS_W // _GCH
_TOKS_W = N // _NW


def _router_body(x_ref, w_ref, b_ref, ptok_ref, desc_ref, aux_ref):
    x = x_ref[...]
    logits = jnp.dot(x, w_ref[...], preferred_element_type=jnp.float32)
    logits = logits + b_ref[...]
    nf = jnp.float32(N)

    lg = logits[:, 0:8]
    lane = lax.broadcasted_iota(jnp.int32, (N, 8), 1)
    segid = jnp.where(lane < MG, 0, jnp.where(lane < MG + MAC, 1, 2))
    neg = jnp.float32(-1e30)
    mxs = [jnp.max(jnp.where(segid == t, lg, neg), axis=1, keepdims=True)
           for t in range(3)]
    mxfull = jnp.where(segid == 0, mxs[0],
                       jnp.where(segid == 1, mxs[1], mxs[2]))
    ex = jnp.exp(lg - mxfull)
    sums = [jnp.sum(jnp.where(segid == t, ex, 0.0), axis=1, keepdims=True)
            for t in range(3)]
    sfull = jnp.where(segid == 0, sums[0],
                      jnp.where(segid == 1, sums[1], sums[2]))
    p = ex / sfull

    lanew = (lane - jnp.where(segid == 0, 0,
                              jnp.where(segid == 1, MG, MG + MAC))
             ).astype(jnp.float32)
    big = jnp.float32(1e9)
    ams = [jnp.min(jnp.where((segid == t) & (lg == mxfull), lanew, big),
                   axis=1, keepdims=True) for t in range(3)]
    topi = ams[0].astype(jnp.int32)
    ids = topi * MAC + jnp.where(topi == 1, ams[2], ams[1]).astype(jnp.int32)

    maskfull = jnp.where(segid == 0, 1.0,
                         jnp.where(segid == 1, (topi == 0).astype(jnp.float32),
                                   (topi == 1).astype(jnp.float32)))
    colsum = jnp.sum(p * maskfull, axis=0, keepdims=True)

    lane = lax.broadcasted_iota(jnp.int32, (N, 8), 1)
    oh = (ids == lane).astype(jnp.float32)
    s = oh
    k = 1
    while k < N:
        s = s + jnp.concatenate([jnp.zeros((k, 8), jnp.float32), s[:N - k]],
                                axis=0)
        k *= 2
    counts = s[N - 1:N, :]
    ranks = jnp.sum(oh * (s - 1.0), axis=1, keepdims=True)
    nbf = jnp.ceil(counts * (1.0 / B))
    acap = nbf * B
    ac = acap
    cumnb = nbf
    for kk in (1, 2, 4):
        zz = jnp.zeros((1, kk), jnp.float32)
        ac = ac + jnp.concatenate([zz, ac[:, :8 - kk]], axis=1)
        cumnb = cumnb + jnp.concatenate([zz, cumnb[:, :8 - kk]], axis=1)
    astart = ac - acap
    base = jnp.sum(oh * astart, axis=1, keepdims=True)
    ptok_ref[...] = (base + ranks).astype(jnp.int32)

    lane8 = lax.broadcasted_iota(jnp.int32, (1, 8), 1)

    cnt0 = jnp.sum(jnp.where(lane8 < MAC, counts, 0.0))
    cnt1 = jnp.sum(jnp.where((lane8 >= MAC) & (lane8 < 2 * MAC), counts, 0.0))
    d0 = jnp.maximum(cnt0, 1.0)
    d1 = jnp.maximum(cnt1, 1.0)
    sc_meta = ALPHA * MG / (nf * nf)
    sc0 = jnp.where(cnt0 > 0.0, ALPHA * MAC / (d0 * d0), 0.0)
    sc1 = jnp.where(cnt1 > 0.0, ALPHA * MAC / (d1 * d1), 0.0)
    scale = jnp.where(lane8 < MG, sc_meta,
                      jnp.where(lane8 < MG + MAC, sc0, sc1))
    aux = jnp.sum(colsum * colsum * scale)

    cum_c = [jnp.sum(jnp.where(lane8 == c, cumnb, 0.0)) for c in range(G)]
    exc_c = [jnp.sum(jnp.where(lane8 == c, cumnb - nbf, 0.0)) for c in range(G)]
    ast_c = [jnp.sum(jnp.where(lane8 == c, astart, 0.0)) * (1.0 / B)
             for c in range(G)]
    total_nb = jnp.sum(nbf)
    jlane = lax.broadcasted_iota(jnp.int32, (1, 16), 1).astype(jnp.float32)
    jc = jnp.minimum(jlane, total_nb - 1.0)
    gid = jnp.zeros((1, 16), jnp.float32)
    for c in range(G):
        gid = gid + jnp.where(jc >= cum_c[c], 1.0, 0.0)
    kj = jc
    rowj = jnp.zeros((1, 16), jnp.float32)
    for c in range(G):
        sel = (gid == c)
        kj = kj - jnp.where(sel, exc_c[c], 0.0)
        rowj = rowj + jnp.where(sel, ast_c[c], 0.0)
    desc_ref[...] = jnp.concatenate(
        [gid, rowj + kj, jnp.where(jlane < total_nb, 1.0, 0.0)],
        axis=0).astype(jnp.int32)
    aux_ref[...] = jnp.full((8, 128), aux, jnp.float32)


def _router(x, wcat, bcat):
    return pl.pallas_call(
        _router_body,
        out_shape=[
            jax.ShapeDtypeStruct((N, 1), jnp.int32),
            jax.ShapeDtypeStruct((3, 16), jnp.int32),
            jax.ShapeDtypeStruct((8, 128), jnp.float32),
        ],
    )(x, wcat, bcat)


@functools.lru_cache(maxsize=1)
def _sc_gathers():
    mesh = plsc.VectorSubcoreMesh(core_axis_name="c", subcore_axis_name="s",
                                  num_cores=_NC, num_subcores=_NS)

    @functools.partial(
        pl.kernel,
        out_type=jax.ShapeDtypeStruct((NPAD, D), jnp.float32),
        mesh=mesh,
        scratch_types=[
            pltpu.VMEM((_TOKS_W,), jnp.int32),
            pltpu.VMEM((_TOKS_W, D), jnp.float32),
            pltpu.SemaphoreType.DMA,
        ],
    )
    def scatter_in(x_hbm, idx_hbm, out_hbm, idx_v, rows_v, sem):
        wid = lax.axis_index("s") * _NC + lax.axis_index("c")
        base = wid * _TOKS_W
        pltpu.sync_copy(idx_hbm.at[pl.ds(base, _TOKS_W)], idx_v)
        pltpu.sync_copy(x_hbm.at[pl.ds(base, _TOKS_W)], rows_v)
        pltpu.async_copy(rows_v, out_hbm.at[idx_v], sem).wait()

    @functools.partial(
        pl.kernel,
        out_type=jax.ShapeDtypeStruct((N, O), jnp.float32),
        mesh=mesh,
        scratch_types=[
            pltpu.VMEM((_TOKS_W,), jnp.int32),
            pltpu.VMEM((_TOKS_W, O), jnp.float32),
            pltpu.SemaphoreType.DMA,
        ],
    )
    def gather_out(tab_hbm, idx_hbm, out_hbm, idx_v, rows_v, sem):
        wid = lax.axis_index("s") * _NC + lax.axis_index("c")
        base = wid * _TOKS_W
        pltpu.sync_copy(idx_hbm.at[pl.ds(base, _TOKS_W)], idx_v)
        pltpu.async_copy(tab_hbm.at[idx_v], rows_v, sem).wait()
        pltpu.sync_copy(rows_v, out_hbm.at[pl.ds(base, _TOKS_W)])

    return scatter_in, gather_out


def _ffn_body(g_ref, r_ref, v_ref, xs_ref, w1_ref, b1_ref, w2_ref, b2_ref,
              w3_ref, b3_ref, out_ref):
    i = pl.program_id(0)
    e = pl.program_id(1)
    valid = v_ref[i] == 1

    @pl.when(valid)
    def _():
        xb = xs_ref[...]
        h = jnp.dot(xb, w1_ref[0, 0], preferred_element_type=jnp.float32)
        h = jnp.maximum(h + b1_ref[0, 0], 0.0)
        h2 = jnp.dot(h, w2_ref[0, 0], preferred_element_type=jnp.float32)
        h2 = jnp.maximum(h2 + b2_ref[0, 0] + xb, 0.0)
        oe = jnp.dot(h2, w3_ref[0, 0], preferred_element_type=jnp.float32)
        oe = (oe + b3_ref[0, 0]) * (1.0 / MIC)

        @pl.when(e == 0)
        def _():
            out_ref[...] = oe

        @pl.when(e > 0)
        def _():
            out_ref[...] += oe


def _ffn(blk_gid, blk_row, blk_val, xs_pad, fc1_W, b1r, fc2_W, b2r, fc3_W, b3r):
    def _e_eff(e, v, i):
        return jnp.where(v[i] == 1, e, MIC - 1)

    grid_spec = pltpu.PrefetchScalarGridSpec(
        num_scalar_prefetch=3,
        grid=(NBMAX, MIC),
        in_specs=[
            pl.BlockSpec((B, D), lambda i, e, g, r, v: (r[i], 0)),
            pl.BlockSpec((1, 1, D, H),
                         lambda i, e, g, r, v: (g[i], _e_eff(e, v, i), 0, 0)),
            pl.BlockSpec((1, 1, H),
                         lambda i, e, g, r, v: (g[i] * MIC + _e_eff(e, v, i), 0, 0)),
            pl.BlockSpec((1, 1, H, H),
                         lambda i, e, g, r, v: (g[i], _e_eff(e, v, i), 0, 0)),
            pl.BlockSpec((1, 1, H),
                         lambda i, e, g, r, v: (g[i] * MIC + _e_eff(e, v, i), 0, 0)),
            pl.BlockSpec((1, 1, H, O),
                         lambda i, e, g, r, v: (g[i], _e_eff(e, v, i), 0, 0)),
            pl.BlockSpec((1, 1, O),
                         lambda i, e, g, r, v: (g[i] * MIC + _e_eff(e, v, i), 0, 0)),
        ],
        out_specs=pl.BlockSpec((B, O), lambda i, e, g, r, v: (r[i], 0)),
    )
    return pl.pallas_call(
        _ffn_body,
        grid_spec=grid_spec,
        out_shape=jax.ShapeDtypeStruct((NPAD, O), jnp.float32),
        compiler_params=pltpu.CompilerParams(
            dimension_semantics=("arbitrary", "arbitrary")),
    )(blk_gid, blk_row, blk_val, xs_pad, fc1_W, b1r, fc2_W, b2r, fc3_W, b3r)


def kernel(x, meta_W, meta_b, macro_W, macro_b,
           fc1_W, fc1_b, fc2_W, fc2_b, fc3_W, fc3_b):
    wcat = jnp.concatenate(
        [meta_W, macro_W[0], macro_W[1],
         jnp.zeros((D, 128 - MG - MG * MAC), jnp.float32)], axis=1)
    bcat = jnp.concatenate(
        [meta_b, macro_b[0], macro_b[1],
         jnp.zeros((128 - MG - MG * MAC,), jnp.float32)])[None, :]

    ptok2d, desc, aux2d = _router(x, wcat, bcat)
    p_tok = ptok2d[:, 0]
    aux = aux2d[0, 0]
    blk_gid = desc[0, :NBMAX]
    blk_row = desc[1, :NBMAX]
    blk_val = desc[2, :NBMAX]

    scatter_in, gather_out = _sc_gathers()
    xs_pad = scatter_in(x, p_tok)

    b1r = fc1_b.reshape(G * MIC, 1, H)
    b2r = fc2_b.reshape(G * MIC, 1, H)
    b3r = fc3_b.reshape(G * MIC, 1, O)
    out_pad = _ffn(blk_gid, blk_row, blk_val, xs_pad,
                   fc1_W, b1r, fc2_W, b2r, fc3_W, b3r)

    final = gather_out(out_pad, p_tok)
    return final, aux
